# baseline (speedup 1.0000x reference)
import functools
import math

import jax
import jax.numpy as jnp
from jax import lax
from jax.experimental import pallas as pl
from jax.experimental.pallas import tpu as pltpu
from jax.experimental.pallas import tpu_sc as plsc

_D = 64
_SCALE = math.sqrt(_D)
_B0W = 128
_B1C = 4
_NCH = 50


def _emb_body(idxt_hbm, table_hbm, out_hbm, idx_v, gbuf0, gbuf1, tbuf,
              g_sems, w_sem, *, nc, b1):
    wid = lax.axis_index("s") * nc + lax.axis_index("c")
    col0 = wid * _B0W
    gbufs = (gbuf0, gbuf1)

    pltpu.sync_copy(idxt_hbm.at[:, pl.ds(col0, _B0W)], idx_v)

    @pl.loop(0, b1)
    def _dbl(r):
        for j in range(_B0W // 16):
            sl = pl.ds(j * 16, 16)
            v = idx_v[r, sl]
            idx_v[r, sl] = v + v

    def fire_gathers(s, c):
        for k in range(_B1C):
            pltpu.async_copy(
                table_hbm.at[idx_v.at[c * _B1C + k]],
                gbufs[s].at[k],
                g_sems.at[s],
            )

    def wait_gathers(s):
        for k in range(_B1C):
            pltpu.make_async_copy(
                table_hbm.at[idx_v.at[0]], gbufs[s].at[k], g_sems.at[s]
            ).wait()

    def wait_writes():
        for k in range(_B1C):
            pltpu.make_async_copy(
                tbuf.at[k], out_hbm.at[0, :, pl.ds(col0, _B0W)], w_sem
            ).wait()

    lane = lax.broadcasted_iota(jnp.int32, (16,), 0)

    def transpose_scale(s):
        @pl.loop(0, _B1C * (_B0W // 16))
        def _blk(i):
            k = i // (_B0W // 16)
            b0b = (i % (_B0W // 16)) * 16
            row_idx = b0b + lane
            kvec = jnp.zeros((16,), jnp.int32) + k
            for dcol in range(_D // 16):
                for j in range(16):
                    d = dcol * 16 + j
                    col = plsc.load_gather(
                        gbufs[s],
                        [kvec, row_idx, jnp.zeros((16,), jnp.int32) + d])
                    tbuf[k, d, pl.ds(b0b, 16)] = col * _SCALE

    def fire_writes(c):
        for k in range(_B1C):
            pltpu.async_copy(
                tbuf.at[k],
                out_hbm.at[c * _B1C + k, :, pl.ds(col0, _B0W)],
                w_sem,
            )

    fire_gathers(0, 0)

    @pl.loop(0, _NCH)
    def _chunk(c):
        s = lax.rem(c, 2)
        for sv in range(2):
            @pl.when(s == sv)
            def _go():
                @pl.when(c + 1 < _NCH)
                def _pref():
                    fire_gathers(1 - sv, c + 1)
                wait_gathers(sv)

                @pl.when(c > 0)
                def _drain():
                    wait_writes()
                transpose_scale(sv)
                fire_writes(c)

    wait_writes()


def kernel(x, table):
    b0, b1 = x.shape
    idxt = x.T.astype(jnp.int32)
    tab = jnp.pad(table, ((0, 0), (0, _D))).reshape(2 * table.shape[0], _D)

    info = plsc.get_sparse_core_info()
    nc, ns = info.num_cores, info.num_subcores

    mesh = plsc.VectorSubcoreMesh(core_axis_name="c", subcore_axis_name="s")
    emb = pl.kernel(
        functools.partial(_emb_body, nc=nc, b1=b1),
        out_type=jax.ShapeDtypeStruct((b1, _D, b0), jnp.float32),
        mesh=mesh,
        compiler_params=pltpu.CompilerParams(
            use_tc_tiling_on_sc=False, needs_layout_passes=False),
        scratch_types=[
            pltpu.VMEM((b1, _B0W), jnp.int32),
            pltpu.VMEM((_B1C, _B0W, _D), jnp.float32),
            pltpu.VMEM((_B1C, _B0W, _D), jnp.float32),
            pltpu.VMEM((_B1C, _D, _B0W), jnp.float32),
            pltpu.SemaphoreType.DMA((2,)),
            pltpu.SemaphoreType.DMA,
        ],
    )
    out2 = emb(idxt, tab)
    return jnp.transpose(out2, (2, 0, 1))


# speedup vs baseline: 2.3806x; 2.3806x over previous
import functools
import math

import jax
import jax.numpy as jnp
from jax import lax
from jax.experimental import pallas as pl
from jax.experimental.pallas import tpu as pltpu
from jax.experimental.pallas import tpu_sc as plsc

_D = 64
_SCALE = math.sqrt(_D)
_IW = 40
_R0 = 4
_GPC = 20
_NBUF = 2


def _emb_body(idx_hbm, table_hbm, out_hbm, idx_v, buf0, buf1, g_sems, w_sems,
              *, nc, w_rows, b1, n_chunks):
    wid = lax.axis_index("s") * nc + lax.axis_index("c")
    base = wid * w_rows
    fbase = base * b1
    rpc = _R0 * b1
    bufs = (buf0, buf1)

    pltpu.sync_copy(idx_hbm.at[pl.ds(fbase, w_rows * b1)], idx_v)

    def fire_gathers(s, g):
        for k in range(_GPC):
            pltpu.async_copy(
                table_hbm.at[idx_v.at[pl.ds((g * _GPC + k) * _IW, _IW)]],
                bufs[s].at[pl.ds(k * _IW, _IW)],
                g_sems.at[s],
            )

    def wait_gathers(s):
        for k in range(_GPC):
            pltpu.make_async_copy(
                table_hbm.at[idx_v.at[pl.ds(0, _IW)]],
                bufs[s].at[pl.ds(k * _IW, _IW)],
                g_sems.at[s],
            ).wait()

    def scale(s):
        @pl.loop(0, rpc)
        def _rows(r):
            for j in range(_D // 16):
                sl = pl.ds(j * 16, 16)
                bufs[s][r, sl] = bufs[s][r, sl] * _SCALE

    def fire_write(s, g):
        pltpu.async_copy(
            bufs[s], out_hbm.at[pl.ds(fbase + g * rpc, rpc), pl.ds(0, _D)],
            w_sems.at[s])

    def wait_write(s):
        pltpu.make_async_copy(
            bufs[s], out_hbm.at[pl.ds(fbase, rpc), pl.ds(0, _D)],
            w_sems.at[s]).wait()

    fire_gathers(0, 0)
    fire_gathers(1, 1)

    @pl.loop(0, n_chunks // 2 - 1)
    def _steady(g2):
        c0 = g2 * 2
        for s in range(_NBUF):
            wait_gathers(s)
            scale(s)
            fire_write(s, c0 + s)
            wait_write(s)
            fire_gathers(s, c0 + s + 2)

    for s in range(_NBUF):
        wait_gathers(s)
        scale(s)
        fire_write(s, n_chunks - 2 + s)
    for s in range(_NBUF):
        wait_write(s)


def kernel(x, table):
    b0, b1 = x.shape
    idx = x.reshape(b0 * b1).astype(jnp.int32) * 2
    tab = jnp.pad(table, ((0, 0), (0, _D))).reshape(2 * table.shape[0], _D)

    info = plsc.get_sparse_core_info()
    nc, ns = info.num_cores, info.num_subcores
    nw = nc * ns
    w_rows = b0 // nw
    n_chunks = w_rows // _R0

    mesh = plsc.VectorSubcoreMesh(core_axis_name="c", subcore_axis_name="s")
    emb = pl.kernel(
        functools.partial(_emb_body, nc=nc, w_rows=w_rows, b1=b1,
                          n_chunks=n_chunks),
        out_type=jax.ShapeDtypeStruct((b0 * b1, 2 * _D), jnp.float32),
        mesh=mesh,
        compiler_params=pltpu.CompilerParams(
            use_tc_tiling_on_sc=False, needs_layout_passes=False),
        scratch_types=[
            pltpu.VMEM((b0 * b1 // nw,), jnp.int32),
            pltpu.VMEM((_R0 * b1, _D), jnp.float32),
            pltpu.VMEM((_R0 * b1, _D), jnp.float32),
            pltpu.SemaphoreType.DMA((_NBUF,)),
            pltpu.SemaphoreType.DMA((_NBUF,)),
        ],
    )
    out = emb(idx, tab)
    return out.reshape(b0, b1, 2 * _D)[:, :, :_D]
